# Initial kernel scaffold; baseline (speedup 1.0000x reference)
import jax
import jax.numpy as jnp
from jax.experimental import pallas as pl


def kernel(x, table, W, b):
    raise NotImplementedError("write your pallas kernel here")


# speedup vs baseline: 1.5693x; 1.5693x over previous
import functools

import jax
import jax.numpy as jnp
from jax import lax
from jax.experimental import pallas as pl
from jax.experimental.pallas import tpu as pltpu
from jax.experimental.pallas import tpu_sc as plsc

_VOCAB = 100000
_EMBED = 16
_BATCH = 1024
_HIST = 200

_NC, _NS = 2, 16
_NW = _NC * _NS
_ROWS_W = _BATCH // _NW
_CHUNK = 100
_CPR = _HIST // _CHUNK
_GROWS = 4
_GCHUNKS = _GROWS * _CPR
_NGROUPS = _ROWS_W // _GROWS


def _pool_body(x_hbm, table_hbm, out_hbm, idx_v, buf_v, pooled_v, sem):
    wid = lax.axis_index("s") * _NC + lax.axis_index("c")
    pltpu.sync_copy(x_hbm.at[wid], idx_v)

    def group(g, carry):
        copies = []
        for k in range(_GCHUNKS):
            c = g * _GCHUNKS + k
            copies.append(
                pltpu.async_copy(
                    table_hbm.at[idx_v.at[c]],
                    buf_v.at[pl.ds(k * _CHUNK, _CHUNK)],
                    sem,
                )
            )
        for cp in copies:
            cp.wait()
        for r in range(_GROWS):
            base = r * _HIST

            def add4(j, acc, base=base):
                o = base + j * 4
                return acc + (
                    (buf_v[o] + buf_v[o + 1]) + (buf_v[o + 2] + buf_v[o + 3])
                )

            acc = lax.fori_loop(
                0, _HIST // 4, add4, jnp.zeros((_EMBED,), jnp.float32)
            )
            pooled_v[g * _GROWS + r] = acc * (1.0 / _HIST)
        return carry

    lax.fori_loop(0, _NGROUPS, group, 0)
    pltpu.sync_copy(pooled_v, out_hbm.at[pl.ds(wid * _ROWS_W, _ROWS_W)])


@functools.partial(
    pl.kernel,
    out_type=jax.ShapeDtypeStruct((_BATCH, _EMBED), jnp.float32),
    mesh=plsc.VectorSubcoreMesh(core_axis_name="c", subcore_axis_name="s"),
    scratch_types=[
        pltpu.VMEM((_ROWS_W * _CPR, _CHUNK), jnp.int32),
        pltpu.VMEM((_GCHUNKS * _CHUNK, _EMBED), jnp.float32),
        pltpu.VMEM((_ROWS_W, _EMBED), jnp.float32),
        pltpu.SemaphoreType.DMA,
    ],
    compiler_params=pltpu.CompilerParams(use_tc_tiling_on_sc=False),
)
def _pool(x_hbm, table_hbm, out_hbm, idx_v, buf_v, pooled_v, sem):
    _pool_body(x_hbm, table_hbm, out_hbm, idx_v, buf_v, pooled_v, sem)


_TV = 2048


def _mm_body(p_ref, w_ref, b_ref, o_ref):
    o_ref[...] = (
        jnp.dot(p_ref[...], w_ref[...], preferred_element_type=jnp.float32)
        + b_ref[...]
    )


def _project(pooled, W, b2d):
    return pl.pallas_call(
        _mm_body,
        grid=(pl.cdiv(_VOCAB, _TV),),
        in_specs=[
            pl.BlockSpec((_BATCH, _EMBED), lambda v: (0, 0)),
            pl.BlockSpec((_EMBED, _TV), lambda v: (0, v)),
            pl.BlockSpec((1, _TV), lambda v: (0, v)),
        ],
        out_specs=pl.BlockSpec((_BATCH, _TV), lambda v: (0, v)),
        out_shape=jax.ShapeDtypeStruct((_BATCH, _VOCAB), jnp.float32),
    )(pooled, W, b2d)


def kernel(x, table, W, b):
    x_r = x.reshape(_NW, _ROWS_W * _CPR, _CHUNK)
    pooled = _pool(x_r, table)
    return _project(pooled, W, b.reshape(1, _VOCAB))


# speedup vs baseline: 1.5735x; 1.0027x over previous
import functools

import jax
import jax.numpy as jnp
from jax import lax
from jax.experimental import pallas as pl
from jax.experimental.pallas import tpu as pltpu
from jax.experimental.pallas import tpu_sc as plsc

_VOCAB = 100000
_EMBED = 16
_BATCH = 1024
_HIST = 200

_NC, _NS = 2, 16
_NW = _NC * _NS
_ROWS_W = _BATCH // _NW
_CHUNK = 100
_CPR = _HIST // _CHUNK
_GROWS = 4
_GCHUNKS = _GROWS * _CPR
_NGROUPS = _ROWS_W // _GROWS


def _pool_body(x_hbm, table_hbm, out_hbm, idx_v, buf_v, pooled_v, sem):
    wid = lax.axis_index("s") * _NC + lax.axis_index("c")
    pltpu.sync_copy(x_hbm.at[wid], idx_v)

    def group(g, carry):
        copies = []
        for k in range(_GCHUNKS):
            c = g * _GCHUNKS + k
            copies.append(
                pltpu.async_copy(
                    table_hbm.at[idx_v.at[c]],
                    buf_v.at[pl.ds(k * _CHUNK, _CHUNK)],
                    sem,
                )
            )
        for cp in copies:
            cp.wait()
        for r in range(_GROWS):
            base = r * _HIST

            def add4(j, acc, base=base):
                o = base + j * 4
                return acc + (
                    (buf_v[o] + buf_v[o + 1]) + (buf_v[o + 2] + buf_v[o + 3])
                )

            acc = lax.fori_loop(
                0, _HIST // 4, add4, jnp.zeros((_EMBED,), jnp.float32)
            )
            pooled_v[g * _GROWS + r] = acc * (1.0 / _HIST)
        return carry

    lax.fori_loop(0, _NGROUPS, group, 0)
    pltpu.sync_copy(pooled_v, out_hbm.at[pl.ds(wid * _ROWS_W, _ROWS_W)])


@functools.partial(
    pl.kernel,
    out_type=jax.ShapeDtypeStruct((_BATCH, _EMBED), jnp.float32),
    mesh=plsc.VectorSubcoreMesh(core_axis_name="c", subcore_axis_name="s"),
    scratch_types=[
        pltpu.VMEM((_ROWS_W * _CPR, _CHUNK), jnp.int32),
        pltpu.VMEM((_GCHUNKS * _CHUNK, _EMBED), jnp.float32),
        pltpu.VMEM((_ROWS_W, _EMBED), jnp.float32),
        pltpu.SemaphoreType.DMA,
    ],
    compiler_params=pltpu.CompilerParams(use_tc_tiling_on_sc=False),
)
def _pool(x_hbm, table_hbm, out_hbm, idx_v, buf_v, pooled_v, sem):
    _pool_body(x_hbm, table_hbm, out_hbm, idx_v, buf_v, pooled_v, sem)


_TV = 4096


def _mm_body(p_ref, w_ref, b_ref, o_ref):
    o_ref[...] = (
        jnp.dot(p_ref[...], w_ref[...], preferred_element_type=jnp.float32)
        + b_ref[...]
    )


def _project(pooled, W, b2d):
    return pl.pallas_call(
        _mm_body,
        grid=(pl.cdiv(_VOCAB, _TV),),
        in_specs=[
            pl.BlockSpec((_BATCH, _EMBED), lambda v: (0, 0)),
            pl.BlockSpec((_EMBED, _TV), lambda v: (0, v)),
            pl.BlockSpec((1, _TV), lambda v: (0, v)),
        ],
        out_specs=pl.BlockSpec((_BATCH, _TV), lambda v: (0, v)),
        out_shape=jax.ShapeDtypeStruct((_BATCH, _VOCAB), jnp.float32),
    )(pooled, W, b2d)


def kernel(x, table, W, b):
    x_r = x.reshape(_NW, _ROWS_W * _CPR, _CHUNK)
    pooled = _pool(x_r, table)
    return _project(pooled, W, b.reshape(1, _VOCAB))


# speedup vs baseline: 1.5930x; 1.0123x over previous
import functools

import jax
import jax.numpy as jnp
from jax import lax
from jax.experimental import pallas as pl
from jax.experimental.pallas import tpu as pltpu
from jax.experimental.pallas import tpu_sc as plsc

_VOCAB = 100000
_EMBED = 16
_BATCH = 1024
_HIST = 200

_NC, _NS = 2, 16
_NW = _NC * _NS
_ROWS_W = _BATCH // _NW
_CHUNK = 100
_CPR = _HIST // _CHUNK
_GROWS = 4
_GCHUNKS = _GROWS * _CPR
_NGROUPS = _ROWS_W // _GROWS


def _pool_body(x_hbm, table_hbm, out_hbm, idx_v, buf_a, buf_b, pooled_v,
               sem_a, sem_b):
    wid = lax.axis_index("s") * _NC + lax.axis_index("c")
    pltpu.sync_copy(x_hbm.at[wid], idx_v)

    def fire(g, buf, sem):
        for k in range(_GCHUNKS):
            c = g * _GCHUNKS + k
            pltpu.async_copy(
                table_hbm.at[idx_v.at[c]],
                buf.at[pl.ds(k * _CHUNK, _CHUNK)],
                sem,
            )

    def drain(buf, sem):
        for k in range(_GCHUNKS):
            pltpu.make_async_copy(
                table_hbm.at[idx_v.at[k]],
                buf.at[pl.ds(k * _CHUNK, _CHUNK)],
                sem,
            ).wait()

    def acc_group(g, buf):
        for r in range(_GROWS):
            base = r * _HIST

            def add8(j, acc, base=base, buf=buf):
                o = base + j * 8
                return acc + (
                    ((buf[o] + buf[o + 1]) + (buf[o + 2] + buf[o + 3]))
                    + ((buf[o + 4] + buf[o + 5]) + (buf[o + 6] + buf[o + 7]))
                )

            acc = lax.fori_loop(
                0, _HIST // 8, add8, jnp.zeros((_EMBED,), jnp.float32),
                unroll=2,
            )
            pooled_v[g * _GROWS + r] = acc * (1.0 / _HIST)

    fire(0, buf_a, sem_a)

    def pair_body(p, carry):
        g0 = 2 * p
        fire(g0 + 1, buf_b, sem_b)
        drain(buf_a, sem_a)
        acc_group(g0, buf_a)

        @pl.when(p < _NGROUPS // 2 - 1)
        def _():
            fire(g0 + 2, buf_a, sem_a)

        drain(buf_b, sem_b)
        acc_group(g0 + 1, buf_b)
        return carry

    lax.fori_loop(0, _NGROUPS // 2, pair_body, 0)
    pltpu.sync_copy(pooled_v, out_hbm.at[pl.ds(wid * _ROWS_W, _ROWS_W)])


@functools.partial(
    pl.kernel,
    out_type=jax.ShapeDtypeStruct((_BATCH, _EMBED), jnp.float32),
    mesh=plsc.VectorSubcoreMesh(core_axis_name="c", subcore_axis_name="s"),
    scratch_types=[
        pltpu.VMEM((_ROWS_W * _CPR, _CHUNK), jnp.int32),
        pltpu.VMEM((_GCHUNKS * _CHUNK, _EMBED), jnp.float32),
        pltpu.VMEM((_GCHUNKS * _CHUNK, _EMBED), jnp.float32),
        pltpu.VMEM((_ROWS_W, _EMBED), jnp.float32),
        pltpu.SemaphoreType.DMA,
        pltpu.SemaphoreType.DMA,
    ],
    compiler_params=pltpu.CompilerParams(use_tc_tiling_on_sc=False),
)
def _pool(x_hbm, table_hbm, out_hbm, idx_v, buf_a, buf_b, pooled_v,
          sem_a, sem_b):
    _pool_body(x_hbm, table_hbm, out_hbm, idx_v, buf_a, buf_b, pooled_v,
               sem_a, sem_b)


_TV = 4096


def _mm_body(p_ref, w_ref, b_ref, o_ref):
    o_ref[...] = (
        jnp.dot(p_ref[...], w_ref[...], preferred_element_type=jnp.float32)
        + b_ref[...]
    )


def _project(pooled, W, b2d):
    return pl.pallas_call(
        _mm_body,
        grid=(pl.cdiv(_VOCAB, _TV),),
        in_specs=[
            pl.BlockSpec((_BATCH, _EMBED), lambda v: (0, 0)),
            pl.BlockSpec((_EMBED, _TV), lambda v: (0, v)),
            pl.BlockSpec((1, _TV), lambda v: (0, v)),
        ],
        out_specs=pl.BlockSpec((_BATCH, _TV), lambda v: (0, v)),
        out_shape=jax.ShapeDtypeStruct((_BATCH, _VOCAB), jnp.float32),
    )(pooled, W, b2d)


def kernel(x, table, W, b):
    x_r = x.reshape(_NW, _ROWS_W * _CPR, _CHUNK)
    pooled = _pool(x_r, table)
    return _project(pooled, W, b.reshape(1, _VOCAB))
